# baseline (speedup 1.0000x reference)
import functools

import jax
import jax.numpy as jnp
from jax import lax
from jax.experimental import pallas as pl
from jax.experimental.pallas import tpu as pltpu
from jax.experimental.pallas import tpu_sc as plsc

_BATCH = 4
_SEQ = 8192
_D = 1024
_B = _BATCH * _SEQ
_NC = 2
_NS = 16
_NW = _NC * _NS
_BPW = _B // _NW
_G = 16
_NGRP = _BPW // _G
_LAG = 2


def _emb_body(idx_hbm, table_hbm, out_hbm, idx_v, sem):
    wid = lax.axis_index("s") * _NC + lax.axis_index("c")
    base = wid * _BPW
    pltpu.sync_copy(idx_hbm.at[pl.ds(base, _BPW)], idx_v)

    def issue(g):
        v = idx_v[pl.ds(g * _G, _G)]
        for lane in range(_G):
            pltpu.async_copy(
                table_hbm.at[pl.ds(v[lane], 1)],
                out_hbm.at[pl.ds(base + g * _G + lane, 1)],
                sem,
            )

    def drain_group():
        for _ in range(_G):
            pltpu.make_async_copy(
                table_hbm.at[pl.ds(0, 1)], out_hbm.at[pl.ds(base, 1)], sem
            ).wait()

    def body(g, carry):
        issue(g)

        @pl.when(g >= _LAG)
        def _():
            drain_group()

        return carry

    lax.fori_loop(0, _NGRP, body, 0)
    for _ in range(_LAG):
        drain_group()


_emb_call = functools.partial(
    pl.kernel,
    out_type=jax.ShapeDtypeStruct((_B, _D), jnp.float32),
    mesh=plsc.VectorSubcoreMesh(core_axis_name="c", subcore_axis_name="s"),
    compiler_params=pltpu.CompilerParams(needs_layout_passes=False),
    scratch_types=[
        pltpu.VMEM((_BPW,), jnp.int32),
        pltpu.SemaphoreType.DMA,
    ],
)(_emb_body)


def kernel(positions, embedding_table):
    idx = positions.astype(jnp.int32).reshape(_B)
    out = _emb_call(idx, embedding_table)
    return out.reshape(_BATCH, _SEQ, _D)


# speedup vs baseline: 36.1129x; 36.1129x over previous
import functools

import jax
import jax.numpy as jnp
from jax import lax
from jax.experimental import pallas as pl
from jax.experimental.pallas import tpu as pltpu
from jax.experimental.pallas import tpu_sc as plsc

_BATCH = 4
_SEQ = 8192
_D = 1024
_B = _BATCH * _SEQ
_NC = 2
_NS = 16
_NW = _NC * _NS
_BPW = _B // _NW
_C = 32
_NCHUNK = _BPW // _C
_NBUF = 2


def _emb_body(idx_hbm, table_hbm, out_hbm, idx_v, rows_v, sem0, sem1):
    sems = (sem0, sem1)
    wid = lax.axis_index("s") * _NC + lax.axis_index("c")
    pltpu.sync_copy(idx_hbm.at[wid], idx_v)

    def start_gather(slot, g):
        for h in range(2):
            pltpu.async_copy(
                table_hbm.at[idx_v.at[g, pl.ds(h * 16, 16)]],
                rows_v.at[slot, pl.ds(h * 16, 16)], sems[slot])

    def wait_gather(slot, g):
        for h in range(2):
            pltpu.make_async_copy(
                table_hbm.at[idx_v.at[g, pl.ds(h * 16, 16)]],
                rows_v.at[slot, pl.ds(h * 16, 16)], sems[slot]
            ).wait()

    for b in range(_NBUF):
        start_gather(b, b)

    n_outer = _NCHUNK // _NBUF

    def outer(it, carry):
        for b in range(_NBUF):
            g = it * _NBUF + b
            wait_gather(b, g)
            pltpu.sync_copy(rows_v.at[b], out_hbm.at[wid, g])
            start_gather(b, g + _NBUF)
        return carry

    lax.fori_loop(0, n_outer - 1, outer, 0)

    for b in range(_NBUF):
        g = (n_outer - 1) * _NBUF + b
        wait_gather(b, g)
        pltpu.sync_copy(rows_v.at[b], out_hbm.at[wid, g])


_emb_call = functools.partial(
    pl.kernel,
    out_type=jax.ShapeDtypeStruct((_NW, _NCHUNK, _C, _D), jnp.float32),
    mesh=plsc.VectorSubcoreMesh(core_axis_name="c", subcore_axis_name="s"),
    scratch_types=[
        pltpu.VMEM((_NCHUNK, _C), jnp.int32),
        pltpu.VMEM((_NBUF, _C, _D), jnp.float32),
        pltpu.SemaphoreType.DMA,
        pltpu.SemaphoreType.DMA,
    ],
)(_emb_body)


def kernel(positions, embedding_table):
    idx = positions.astype(jnp.int32).reshape(_NW, _NCHUNK, _C)
    out = _emb_call(idx, embedding_table)
    return out.reshape(_BATCH, _SEQ, _D)
